# Initial kernel scaffold; baseline (speedup 1.0000x reference)
import jax
import jax.numpy as jnp
from jax.experimental import pallas as pl


def kernel(x, edge_index, W1, b1, W2, b2, W3, b3, gamma, beta, Wc, bc, Wr, br):
    raise NotImplementedError("write your pallas kernel here")


# speedup vs baseline: 10.4688x; 10.4688x over previous
import functools

import jax
import jax.numpy as jnp
from jax import lax
from jax.experimental import pallas as pl
from jax.experimental.pallas import tpu as pltpu
from jax.experimental.pallas import tpu_sc as plsc

N = 10000
NP = 10240
F = 128
E = 320000
O = 40

NC = 2
NS = 16
RPT = NP // NS
RC = 128
EDGES_PER_SC = E // NC
EDGES_PER_TILE = EDGES_PER_SC // NS
SE = 80
NCHUNK = EDGES_PER_TILE // SE

CH = 1000
NBLK = N // CH

_MESH = plsc.VectorSubcoreMesh(core_axis_name="c", subcore_axis_name="s")


@functools.partial(
    pl.kernel,
    out_type=jax.ShapeDtypeStruct((NC, NP, F), jnp.float32),
    mesh=_MESH,
    scratch_types=[
        pltpu.VMEM((SE,), jnp.int32),
        pltpu.VMEM((SE, F), jnp.float32),
        pltpu.VMEM((RC, F), jnp.float32),
        pltpu.VMEM_SHARED((NP, F), jnp.float32),
    ],
)
def _deg_kernel(dst_hbm, ones_hbm, zeros_hbm, out_hbm, di_v, rows_v, stage_v, acc_sh):
    c = lax.axis_index("c")
    s = lax.axis_index("s")
    r0 = s * RPT

    def init_body(j, carry):
        rr = r0 + j * RC

        @pl.when(c == 0)
        def _():
            pltpu.sync_copy(ones_hbm.at[pl.ds(rr, RC)], stage_v)

        @pl.when(c != 0)
        def _():
            pltpu.sync_copy(zeros_hbm.at[pl.ds(rr, RC)], stage_v)

        pltpu.sync_copy(stage_v, acc_sh.at[pl.ds(rr, RC)])
        return carry

    lax.fori_loop(0, RPT // RC, init_body, 0)
    pltpu.sync_copy(ones_hbm.at[pl.ds(0, SE)], rows_v)
    plsc.subcore_barrier()

    e0 = c * EDGES_PER_SC + s * EDGES_PER_TILE

    def body(j, carry):
        pltpu.sync_copy(dst_hbm.at[pl.ds(e0 + j * SE, SE)], di_v)
        pltpu.sync_copy(rows_v, acc_sh.at[di_v], add=True)
        return carry

    lax.fori_loop(0, NCHUNK, body, 0)
    plsc.subcore_barrier()

    def out_body(j, carry):
        rr = r0 + j * RC
        pltpu.sync_copy(acc_sh.at[pl.ds(rr, RC)], stage_v)
        pltpu.sync_copy(stage_v, out_hbm.at[c, pl.ds(rr, RC)])
        return carry

    lax.fori_loop(0, RPT // RC, out_body, 0)


@functools.partial(
    pl.kernel,
    out_type=jax.ShapeDtypeStruct((NC, NP, F), jnp.float32),
    mesh=_MESH,
    scratch_types=[
        pltpu.VMEM((SE,), jnp.int32),
        pltpu.VMEM((SE,), jnp.int32),
        pltpu.VMEM((SE, F), jnp.float32),
        pltpu.VMEM((RC, F), jnp.float32),
        pltpu.VMEM_SHARED((NP, F), jnp.float32),
    ],
)
def _scatter_kernel(g_hbm, src_hbm, dst_hbm, zeros_hbm, out_hbm,
                    si_v, di_v, rows_v, stage_v, acc_sh):
    c = lax.axis_index("c")
    s = lax.axis_index("s")
    r0 = s * RPT

    def init_body(j, carry):
        rr = r0 + j * RC

        @pl.when(c == 0)
        def _():
            pltpu.sync_copy(g_hbm.at[pl.ds(rr, RC)], stage_v)

        @pl.when(c != 0)
        def _():
            pltpu.sync_copy(zeros_hbm.at[pl.ds(rr, RC)], stage_v)

        pltpu.sync_copy(stage_v, acc_sh.at[pl.ds(rr, RC)])
        return carry

    lax.fori_loop(0, RPT // RC, init_body, 0)
    plsc.subcore_barrier()

    e0 = c * EDGES_PER_SC + s * EDGES_PER_TILE

    def body(j, carry):
        b = e0 + j * SE
        pltpu.sync_copy(src_hbm.at[pl.ds(b, SE)], si_v)
        pltpu.sync_copy(dst_hbm.at[pl.ds(b, SE)], di_v)
        pltpu.sync_copy(g_hbm.at[si_v], rows_v)
        pltpu.sync_copy(rows_v, acc_sh.at[di_v], add=True)
        return carry

    lax.fori_loop(0, NCHUNK, body, 0)
    plsc.subcore_barrier()

    def out_body(j, carry):
        rr = r0 + j * RC
        pltpu.sync_copy(acc_sh.at[pl.ds(rr, RC)], stage_v)
        pltpu.sync_copy(stage_v, out_hbm.at[c, pl.ds(rr, RC)])
        return carry

    lax.fori_loop(0, RPT // RC, out_body, 0)


def _tc1_body(x_ref, w_ref, degp_ref, g_ref, dinv_ref):
    d = degp_ref[0][:, 0:1] + degp_ref[1][:, 0:1]
    dv = lax.rsqrt(d)
    h = jnp.dot(x_ref[...], w_ref[...], preferred_element_type=jnp.float32)
    g_ref[...] = h * dv
    dinv_ref[...] = dv


def _tc_mid_body(acc_ref, dinv_ref, b_ref, w_ref, g_ref):
    dv = dinv_ref[...]
    h = (acc_ref[0] + acc_ref[1]) * dv + b_ref[...]
    h = jnp.maximum(h, 0.0)
    g_ref[...] = jnp.dot(h, w_ref[...], preferred_element_type=jnp.float32) * dv


def _tc_h3_body(acc_ref, dinv_ref, b_ref, h_ref, st_ref):
    i = pl.program_id(0)
    h = (acc_ref[0] + acc_ref[1]) * dinv_ref[...] + b_ref[...]
    h = jnp.maximum(h, 0.0)
    h_ref[...] = h
    s1 = jnp.sum(h, axis=0, keepdims=True)
    s2 = jnp.sum(h * h, axis=0, keepdims=True)
    st = jnp.concatenate([s1, s2], axis=0)

    @pl.when(i == 0)
    def _():
        st_ref[...] = st

    @pl.when(i != 0)
    def _():
        st_ref[...] += st


def _tc_head_body(h_ref, st_ref, gam_ref, bet_ref, wc_ref, bc_ref,
                  wr_ref, br_ref, o_ref):
    mean = st_ref[0:1] / float(N)
    var = st_ref[1:2] / float(N) - mean * mean
    xn = (h_ref[...] - mean) * lax.rsqrt(var + 1e-5) * gam_ref[...] + bet_ref[...]
    hc = jnp.dot(xn, wc_ref[...], preferred_element_type=jnp.float32) + bc_ref[...]
    hc = jnp.maximum(hc, 0.0)
    o_ref[...] = jnp.dot(hc, wr_ref[...], preferred_element_type=jnp.float32) + br_ref[...]


def _tc1(x, W1, degp):
    return pl.pallas_call(
        _tc1_body,
        grid=(NBLK,),
        in_specs=[
            pl.BlockSpec((CH, F), lambda i: (i, 0)),
            pl.BlockSpec((F, F), lambda i: (0, 0)),
            pl.BlockSpec((2, CH, F), lambda i: (0, i, 0)),
        ],
        out_specs=[
            pl.BlockSpec((CH, F), lambda i: (i, 0)),
            pl.BlockSpec((CH, 1), lambda i: (i, 0)),
        ],
        out_shape=[
            jax.ShapeDtypeStruct((NP, F), jnp.float32),
            jax.ShapeDtypeStruct((NP, 1), jnp.float32),
        ],
    )(x, W1, degp)


def _tc_mid(acc, dinv, b, W):
    return pl.pallas_call(
        _tc_mid_body,
        grid=(NBLK,),
        in_specs=[
            pl.BlockSpec((2, CH, F), lambda i: (0, i, 0)),
            pl.BlockSpec((CH, 1), lambda i: (i, 0)),
            pl.BlockSpec((1, F), lambda i: (0, 0)),
            pl.BlockSpec((F, F), lambda i: (0, 0)),
        ],
        out_specs=pl.BlockSpec((CH, F), lambda i: (i, 0)),
        out_shape=jax.ShapeDtypeStruct((NP, F), jnp.float32),
    )(acc, dinv, b, W)


def _tc_h3(acc, dinv, b):
    return pl.pallas_call(
        _tc_h3_body,
        grid=(NBLK,),
        in_specs=[
            pl.BlockSpec((2, CH, F), lambda i: (0, i, 0)),
            pl.BlockSpec((CH, 1), lambda i: (i, 0)),
            pl.BlockSpec((1, F), lambda i: (0, 0)),
        ],
        out_specs=[
            pl.BlockSpec((CH, F), lambda i: (i, 0)),
            pl.BlockSpec((2, F), lambda i: (0, 0)),
        ],
        out_shape=[
            jax.ShapeDtypeStruct((N, F), jnp.float32),
            jax.ShapeDtypeStruct((2, F), jnp.float32),
        ],
    )(acc, dinv, b)


def _tc_head(h3, stats, gamma, beta, Wc, bc, Wr, br):
    return pl.pallas_call(
        _tc_head_body,
        grid=(NBLK,),
        in_specs=[
            pl.BlockSpec((CH, F), lambda i: (i, 0)),
            pl.BlockSpec((2, F), lambda i: (0, 0)),
            pl.BlockSpec((1, F), lambda i: (0, 0)),
            pl.BlockSpec((1, F), lambda i: (0, 0)),
            pl.BlockSpec((F, F), lambda i: (0, 0)),
            pl.BlockSpec((1, F), lambda i: (0, 0)),
            pl.BlockSpec((F, O), lambda i: (0, 0)),
            pl.BlockSpec((1, O), lambda i: (0, 0)),
        ],
        out_specs=pl.BlockSpec((CH, O), lambda i: (i, 0)),
        out_shape=jax.ShapeDtypeStruct((N, O), jnp.float32),
    )(h3, stats, gamma, beta, Wc, bc, Wr, br)


def kernel(x, edge_index, W1, b1, W2, b2, W3, b3, gamma, beta, Wc, bc, Wr, br):
    src = edge_index[0]
    dst = edge_index[1]
    zeros = jnp.zeros((NP, F), jnp.float32)
    ones_tab = jnp.ones((NP, F), jnp.float32)

    degp = _deg_kernel(dst, ones_tab, zeros)
    g1, dinv = _tc1(x, W1, degp)
    acc1 = _scatter_kernel(g1, src, dst, zeros)
    g2 = _tc_mid(acc1, dinv, b1.reshape(1, F), W2)
    acc2 = _scatter_kernel(g2, src, dst, zeros)
    g3 = _tc_mid(acc2, dinv, b2.reshape(1, F), W3)
    acc3 = _scatter_kernel(g3, src, dst, zeros)
    h3, stats = _tc_h3(acc3, dinv, b3.reshape(1, F))
    return _tc_head(h3, stats, gamma.reshape(1, F), beta.reshape(1, F),
                    Wc, bc.reshape(1, F), Wr, br.reshape(1, O))
